# baseline (speedup 1.0000x reference)
import jax
import jax.numpy as jnp
from jax.experimental import pallas as pl
from jax.experimental.pallas import tpu as pltpu

DEPTH = 1000
ROWS = 4096 * 26
BLK_R = 1024


def _onehot_body(idx_ref, out_ref):
    idx = idx_ref[0, 0, :].reshape(BLK_R, 1)
    iota = jax.lax.broadcasted_iota(jnp.int32, (BLK_R, DEPTH), 1)
    out_ref[...] = (iota == idx).astype(jnp.float32)


def kernel(inputs):
    idx = inputs.reshape(ROWS // BLK_R, 1, BLK_R).astype(jnp.int32)
    out = pl.pallas_call(
        _onehot_body,
        grid=(ROWS // BLK_R,),
        in_specs=[pl.BlockSpec((1, 1, BLK_R), lambda i: (i, 0, 0))],
        out_specs=pl.BlockSpec((BLK_R, DEPTH), lambda i: (i, 0)),
        out_shape=jax.ShapeDtypeStruct((ROWS, DEPTH), jnp.float32),
    )(idx)
    return out.reshape(4096, 26, DEPTH)


# speedup vs baseline: 1.3875x; 1.3875x over previous
import jax
import jax.numpy as jnp
from jax.experimental import pallas as pl
from jax.experimental.pallas import tpu as pltpu

DEPTH = 1000
B0 = 32


def _onehot_body(idx_ref, out_ref):
    idx = idx_ref[...][:, :, None]
    iota = jax.lax.broadcasted_iota(jnp.int32, (B0, 26, DEPTH), 2)
    out_ref[...] = (iota == idx).astype(jnp.float32)


def kernel(inputs):
    idx = inputs.astype(jnp.int32)
    return pl.pallas_call(
        _onehot_body,
        grid=(4096 // B0,),
        in_specs=[pl.BlockSpec((B0, 26), lambda i: (i, 0))],
        out_specs=pl.BlockSpec((B0, 26, DEPTH), lambda i: (i, 0, 0)),
        out_shape=jax.ShapeDtypeStruct((4096, 26, DEPTH), jnp.float32),
    )(idx)
